# Initial kernel scaffold; baseline (speedup 1.0000x reference)
import jax
import jax.numpy as jnp
from jax.experimental import pallas as pl


def kernel(users, edge_index, edge_weight, user_emb, item_emb):
    raise NotImplementedError("write your pallas kernel here")


# speedup vs baseline: 10.8245x; 10.8245x over previous
import functools

import jax
import jax.numpy as jnp
from jax import lax
from jax.experimental import pallas as pl
from jax.experimental.pallas import tpu as pltpu
from jax.experimental.pallas import tpu_sc as plsc

N_USERS = 50000
N_ITEMS = 50000
N_NODES = N_USERS + N_ITEMS
DIM = 32
HALF = 16
N_EDGES = 1600000
BATCH_USERS = 256

NC = 2
NS = 16

CHB = 5
CH = CHB * 128
NCH = 162
EPC = 3 * CHB
EPT = NCH * CH
TOT = EPT * NS
ROWS_PER_TILE = 6256
ROWS_LAST = N_NODES - 15 * ROWS_PER_TILE

_mesh = plsc.VectorSubcoreMesh(
    core_axis_name="c", subcore_axis_name="s", num_cores=NC, num_subcores=NS
)

_sc_params = pltpu.CompilerParams(
    use_tc_tiling_on_sc=False, needs_layout_passes=False
)


@functools.partial(
    pl.kernel,
    mesh=_mesh,
    out_type=jax.ShapeDtypeStruct((2 * N_NODES, HALF), jnp.float32),
    compiler_params=_sc_params,
    scratch_types=[
        pltpu.VMEM((EPC, 128), jnp.int32),
        pltpu.VMEM((EPC, 128), jnp.int32),
        pltpu.VMEM((EPC, 128), jnp.int32),
        pltpu.VMEM((CH, HALF), jnp.float32),
        pltpu.VMEM((CH, HALF), jnp.float32),
        pltpu.VMEM_SHARED((N_NODES, HALF), jnp.float32),
        pltpu.SemaphoreType.DMA,
        pltpu.SemaphoreType.DMA,
        pltpu.SemaphoreType.DMA,
        pltpu.SemaphoreType.DMA,
        pltpu.SemaphoreType.DMA,
        pltpu.SemaphoreType.DMA,
        pltpu.SemaphoreType.DMA,
        pltpu.SemaphoreType.DMA,
    ],
)
def _layer(epack_hbm, emb_hbm, out_hbm,
           ep0, ep1, ep2, r0, r1, acc,
           isem0, isem1, isem2, gsem0, gsem1, ssem0, ssem1, _spare):
    c = lax.axis_index("c")
    s = lax.axis_index("s")
    off = c * N_NODES
    tb = s * (NCH * EPC)

    eps = (ep0, ep1, ep2)
    isems = (isem0, isem1, isem2)
    rs = (r0, r1)
    gsems = (gsem0, gsem1)
    ssems = (ssem0, ssem1)

    @plsc.parallel_loop(0, CH)
    def _zb(i):
        r0[i, :] = jnp.zeros((16,), jnp.float32)

    base = s * ROWS_PER_TILE
    for z in range(ROWS_PER_TILE // CH):
        pltpu.sync_copy(r0, acc.at[pl.ds(base + z * CH, CH)])
    zrem = ROWS_PER_TILE - (ROWS_PER_TILE // CH) * CH
    zrem_l = ROWS_LAST - (ROWS_PER_TILE // CH) * CH

    @pl.when(s < NS - 1)
    def _():
        pltpu.sync_copy(r0.at[pl.ds(0, zrem)],
                        acc.at[pl.ds(base + 9 * CH, zrem)])

    @pl.when(s == NS - 1)
    def _():
        pltpu.sync_copy(r0.at[pl.ds(0, zrem_l)],
                        acc.at[pl.ds(base + 9 * CH, zrem_l)])

    plsc.subcore_barrier()

    def idx_start(cc, ep, isem):
        pltpu.async_copy(epack_hbm.at[pl.ds(tb + cc * EPC, EPC)], ep, isem)

    def idx_wait(ep, isem):
        pltpu.make_async_copy(epack_hbm.at[pl.ds(0, EPC)], ep, isem).wait()

    def add_off(ep):
        @plsc.parallel_loop(0, CHB * 8)
        def _ao(g):
            sl = pl.ds(lax.mul(lax.rem(g, 8), 16), 16)
            row = lax.shift_right_logical(g, 3)
            ep[row, sl] = ep[row, sl] + off

    def fire_gathers(ep, r, gsem):
        for jb in range(CHB):
            pltpu.async_copy(
                emb_hbm.at[ep.at[jb]], r.at[pl.ds(jb * 128, 128)], gsem
            )

    def drain_rows(r, sem):
        pltpu.make_async_copy(emb_hbm.at[pl.ds(0, CH)], r, sem).wait()

    def scale(ep, r):
        @plsc.parallel_loop(0, CH // 16)
        def _grp(g):
            sl = pl.ds(lax.mul(lax.rem(g, 8), 16), 16)
            row = 2 * CHB + lax.shift_right_logical(g, 3)
            w16 = plsc.bitcast(ep[row, sl], jnp.float32)
            base_i = g * 16
            for k in range(16):
                r[base_i + k, :] = r[base_i + k, :] * w16[k]

    def fire_scatters(ep, r, ssem):
        for jb in range(CHB):
            pltpu.async_copy(
                r.at[pl.ds(jb * 128, 128)], acc.at[ep.at[CHB + jb]],
                ssem, add=True,
            )

    idx_start(0, eps[0], isems[0])
    idx_wait(eps[0], isems[0])
    add_off(eps[0])
    fire_gathers(eps[0], rs[0], gsems[0])
    idx_start(1, eps[1], isems[1])

    def do_chunk(k, cc, i):
        a3, b3, n3 = k % 3, (k + 1) % 3, (k + 2) % 3
        a2, b2 = k % 2, (k + 1) % 2
        idx_wait(eps[b3], isems[b3])
        add_off(eps[b3])
        if k == 0:
            @pl.when(i > 0)
            def _():
                drain_rows(rs[b2], ssems[b2])
        else:
            drain_rows(rs[b2], ssems[b2])
        idx_start(cc + 2, eps[n3], isems[n3])
        fire_gathers(eps[b3], rs[b2], gsems[b2])
        drain_rows(rs[a2], gsems[a2])
        scale(eps[a3], rs[a2])
        fire_scatters(eps[a3], rs[a2], ssems[a2])

    def body(i, _):
        for k in range(6):
            do_chunk(k, 6 * i + k, i)
        return 0

    lax.fori_loop(0, NCH // 6, body, 0)

    drain_rows(rs[0], gsems[0])
    idx_wait(eps[1], isems[1])
    drain_rows(rs[1], ssems[1])

    plsc.subcore_barrier()

    @pl.when(s < NS - 1)
    def _():
        pltpu.sync_copy(
            acc.at[pl.ds(base, ROWS_PER_TILE)],
            out_hbm.at[pl.ds(c * N_NODES + base, ROWS_PER_TILE)],
        )

    @pl.when(s == NS - 1)
    def _():
        pltpu.sync_copy(
            acc.at[pl.ds(base, ROWS_LAST)],
            out_hbm.at[pl.ds(c * N_NODES + base, ROWS_LAST)],
        )


_UPT = BATCH_USERS // (NC * NS)


@functools.partial(
    pl.kernel,
    mesh=_mesh,
    out_type=jax.ShapeDtypeStruct((BATCH_USERS, DIM), jnp.float32),
    compiler_params=_sc_params,
    scratch_types=[
        pltpu.VMEM((_UPT,), jnp.int32),
        pltpu.VMEM((_UPT,), jnp.int32),
        pltpu.VMEM((_UPT, HALF), jnp.float32),
        pltpu.VMEM((_UPT, HALF), jnp.float32),
        pltpu.VMEM((_UPT, HALF), jnp.float32),
        pltpu.VMEM((_UPT, DIM), jnp.float32),
        pltpu.SemaphoreType.DMA,
    ],
)
def _user_mean(users_hbm, usersb_hbm, e0, e1, e2, e3, out_hbm,
               uidx, uidxb, tmp, acca, accb, st, sem):
    t = lax.axis_index("s") * NC + lax.axis_index("c")
    pltpu.sync_copy(users_hbm.at[pl.ds(t * _UPT, _UPT)], uidx)
    pltpu.sync_copy(usersb_hbm.at[pl.ds(t * _UPT, _UPT)], uidxb)
    for li, e in enumerate((e0, e1, e2, e3)):
        pltpu.async_copy(e.at[uidx], tmp, sem).wait()
        for i in range(_UPT):
            if li == 0:
                acca[i, :] = tmp[i, :]
            else:
                acca[i, :] = acca[i, :] + tmp[i, :]
        pltpu.async_copy(e.at[uidxb], tmp, sem).wait()
        for i in range(_UPT):
            if li == 0:
                accb[i, :] = tmp[i, :]
            else:
                accb[i, :] = accb[i, :] + tmp[i, :]
    for i in range(_UPT):
        st[i, 0:HALF] = acca[i, :] * 0.25
        st[i, HALF:DIM] = accb[i, :] * 0.25
    pltpu.sync_copy(st, out_hbm.at[pl.ds(t * _UPT, _UPT)])


_BI = 2000
_BU = 8


def _imean_body(a0, a1, a2, a3, b0, b1, b2, b3, o_ref):
    ia = (a0[...] + a1[...] + a2[...] + a3[...]) * 0.25
    ib = (b0[...] + b1[...] + b2[...] + b3[...]) * 0.25
    o_ref[...] = jnp.concatenate([ia, ib], axis=1)


def _rating_body(u_ref, items_ref, o_ref):
    acc = lax.dot_general(
        u_ref[...], items_ref[...], (((1,), (1,)), ((), ())),
        preferred_element_type=jnp.float32,
    )
    o_ref[...] = 1.0 / (1.0 + jnp.exp(-acc))


def _epilogue(u, e0, e1, e2, e3):
    a_off = N_USERS // _BI
    b_off = (N_NODES + N_USERS) // _BI
    specs = [
        pl.BlockSpec((_BI, HALF), lambda i, o=a_off: (o + i, 0)) for _ in range(4)
    ]
    specs += [
        pl.BlockSpec((_BI, HALF), lambda i, o=b_off: (o + i, 0)) for _ in range(4)
    ]
    items = pl.pallas_call(
        _imean_body,
        grid=(N_ITEMS // _BI,),
        in_specs=specs,
        out_specs=pl.BlockSpec((_BI, DIM), lambda i: (i, 0)),
        out_shape=jax.ShapeDtypeStruct((N_ITEMS, DIM), jnp.float32),
    )(e0, e1, e2, e3, e0, e1, e2, e3)
    return pl.pallas_call(
        _rating_body,
        grid=(BATCH_USERS // _BU,),
        in_specs=[
            pl.BlockSpec((_BU, DIM), lambda i: (i, 0)),
            pl.BlockSpec((N_ITEMS, DIM), lambda i: (0, 0)),
        ],
        out_specs=pl.BlockSpec((_BU, N_ITEMS), lambda i: (i, 0)),
        out_shape=jax.ShapeDtypeStruct((BATCH_USERS, N_ITEMS), jnp.float32),
    )(u, items)


def kernel(users, edge_index, edge_weight, user_emb, item_emb):
    users = users.astype(jnp.int32)
    src = edge_index[0].astype(jnp.int32)
    dst = edge_index[1].astype(jnp.int32)
    w = edge_weight.astype(jnp.float32)

    all_emb = jnp.concatenate([user_emb, item_emb], axis=0)
    e0 = jnp.concatenate([all_emb[:, :HALF], all_emb[:, HALF:]], axis=0)

    pad = TOT - N_EDGES
    srcr = jnp.pad(src, (0, pad)).reshape(NS, NCH, CHB, 128)
    dstr = jnp.pad(dst, (0, pad)).reshape(NS, NCH, CHB, 128)
    wbits = lax.bitcast_convert_type(
        jnp.pad(w, (0, pad)), jnp.int32
    ).reshape(NS, NCH, CHB, 128)
    epack = jnp.concatenate([srcr, dstr, wbits], axis=2)
    epack = epack.reshape(NS * NCH * EPC, 128)
    epack = jnp.pad(epack, ((0, 2 * EPC), (0, 0)))

    e1 = _layer(epack, e0)
    e2 = _layer(epack, e1)
    e3 = _layer(epack, e2)

    u = _user_mean(users, users + N_NODES, e0, e1, e2, e3)
    return _epilogue(u, e0, e1, e2, e3)


# speedup vs baseline: 11.9477x; 1.1038x over previous
import functools

import jax
import jax.numpy as jnp
from jax import lax
from jax.experimental import pallas as pl
from jax.experimental.pallas import tpu as pltpu
from jax.experimental.pallas import tpu_sc as plsc

N_USERS = 50000
N_ITEMS = 50000
N_NODES = N_USERS + N_ITEMS
DIM = 32
HALF = 16
N_EDGES = 1600000
BATCH_USERS = 256

NC = 2
NS = 16

CHB = 5
CH = CHB * 128
NCH = 162
EPT = NCH * CH
TOT = EPT * NS
IDXROWS = TOT // 128
ROWS_PER_TILE = 6256
ROWS_LAST = N_NODES - 15 * ROWS_PER_TILE

_mesh = plsc.VectorSubcoreMesh(
    core_axis_name="c", subcore_axis_name="s", num_cores=NC, num_subcores=NS
)

_sc_params = pltpu.CompilerParams(
    use_tc_tiling_on_sc=False, needs_layout_passes=False
)

_SCRATCH = (
    [pltpu.VMEM((CHB, 128), jnp.int32) for _ in range(3)]
    + [pltpu.VMEM((CHB, 128), jnp.int32) for _ in range(3)]
    + [pltpu.VMEM((CHB, 128), jnp.float32) for _ in range(3)]
    + [pltpu.VMEM((CH, HALF), jnp.float32) for _ in range(2)]
    + [
        pltpu.VMEM((16, HALF), jnp.float32),
        pltpu.VMEM((16, HALF), jnp.float32),
        pltpu.VMEM((16, HALF), jnp.float32),
        pltpu.VMEM((16,), jnp.int32),
        pltpu.VMEM_SHARED((N_NODES, HALF), jnp.float32),
    ]
    + [pltpu.SemaphoreType.DMA for _ in range(7)]
)


@functools.partial(
    pl.kernel,
    mesh=_mesh,
    out_type=(
        jax.ShapeDtypeStruct((2 * N_NODES, HALF), jnp.float32),
        jax.ShapeDtypeStruct((2 * N_NODES, HALF), jnp.float32),
        jax.ShapeDtypeStruct((2 * N_NODES, HALF), jnp.float32),
        jax.ShapeDtypeStruct((2 * BATCH_USERS, HALF), jnp.float32),
    ),
    compiler_params=_sc_params,
    scratch_types=_SCRATCH,
)
def _propagate(src_hbm, dst_hbm, w_hbm, allcat_hbm,
               uie0_hbm, uies_hbm,
               e1_hbm, e2_hbm, esum_hbm, u_hbm,
               sb0, sb1, sb2, db0, db1, db2, wb0, wb1, wb2, r0, r1,
               uta, utb, uto, uidx, acc,
               isem0, isem1, isem2, gsem0, gsem1, ssem0, ssem1):
    c = lax.axis_index("c")
    s = lax.axis_index("s")
    sbufs = (sb0, sb1, sb2)
    dbufs = (db0, db1, db2)
    wbufs = (wb0, wb1, wb2)
    isems = (isem0, isem1, isem2)
    rs = (r0, r1)
    gsems = (gsem0, gsem1)
    ssems = (ssem0, ssem1)
    base = s * ROWS_PER_TILE
    tile_row0 = s * (EPT // 128)

    def zero_acc():
        @plsc.parallel_loop(0, CH)
        def _zb(i):
            r0[i, :] = jnp.zeros((16,), jnp.float32)

        for z in range(ROWS_PER_TILE // CH):
            pltpu.sync_copy(r0, acc.at[pl.ds(base + z * CH, CH)])
        zrem = ROWS_PER_TILE - (ROWS_PER_TILE // CH) * CH
        zrem_l = ROWS_LAST - (ROWS_PER_TILE // CH) * CH

        @pl.when(s < NS - 1)
        def _():
            pltpu.sync_copy(r0.at[pl.ds(0, zrem)],
                            acc.at[pl.ds(base + 9 * CH, zrem)])

        @pl.when(s == NS - 1)
        def _():
            pltpu.sync_copy(r0.at[pl.ds(0, zrem_l)],
                            acc.at[pl.ds(base + 9 * CH, zrem_l)])

    def idx_start(cc, k3):
        rr = tile_row0 + cc * CHB
        pltpu.async_copy(src_hbm.at[pl.ds(rr, CHB)], sbufs[k3], isems[k3])
        pltpu.async_copy(dst_hbm.at[pl.ds(rr, CHB)], dbufs[k3], isems[k3])
        pltpu.async_copy(w_hbm.at[pl.ds(rr, CHB)], wbufs[k3], isems[k3])

    def idx_wait(k3):
        pltpu.make_async_copy(src_hbm.at[pl.ds(0, CHB)], sbufs[k3], isems[k3]).wait()
        pltpu.make_async_copy(dst_hbm.at[pl.ds(0, CHB)], dbufs[k3], isems[k3]).wait()
        pltpu.make_async_copy(w_hbm.at[pl.ds(0, CHB)], wbufs[k3], isems[k3]).wait()

    def add_off(k3, mul, addend):
        sb = sbufs[k3]

        @plsc.parallel_loop(0, CHB * 8)
        def _ao(g):
            sl = pl.ds(lax.mul(lax.rem(g, 8), 16), 16)
            row = lax.shift_right_logical(g, 3)
            v = sb[row, sl]
            if mul == 2:
                v = v + v
            sb[row, sl] = v + addend

    def fire_gathers(table, k3, k2):
        for jb in range(CHB):
            pltpu.async_copy(
                table.at[sbufs[k3].at[jb]],
                rs[k2].at[pl.ds(jb * 128, 128)], gsems[k2],
            )

    def drain_rows(k2, sem):
        pltpu.make_async_copy(
            allcat_hbm.at[pl.ds(0, CH)], rs[k2], sem
        ).wait()

    def scale(k3, k2):
        wbuf = wbufs[k3]
        r = rs[k2]

        @plsc.parallel_loop(0, CH // 16)
        def _grp(g):
            sl = pl.ds(lax.mul(lax.rem(g, 8), 16), 16)
            w16 = wbuf[lax.shift_right_logical(g, 3), sl]
            base_i = g * 16
            for k in range(16):
                r[base_i + k, :] = r[base_i + k, :] * w16[k]

    def fire_scatters(k3, k2):
        for jb in range(CHB):
            pltpu.async_copy(
                rs[k2].at[pl.ds(jb * 128, 128)],
                acc.at[dbufs[k3].at[jb]], ssems[k2], add=True,
            )

    def run_layer(table, mul, addend):
        zero_acc()
        plsc.subcore_barrier()
        idx_start(0, 0)
        idx_wait(0)
        add_off(0, mul, addend)
        fire_gathers(table, 0, 0)
        idx_start(1, 1)

        def do_chunk(k, cc, i):
            a3, b3, n3 = k % 3, (k + 1) % 3, (k + 2) % 3
            a2, b2 = k % 2, (k + 1) % 2
            idx_wait(b3)
            add_off(b3, mul, addend)
            if k == 0:
                @pl.when(i > 0)
                def _():
                    drain_rows(b2, ssems[b2])
            else:
                drain_rows(b2, ssems[b2])
            idx_start(cc + 2, n3)
            fire_gathers(table, b3, b2)
            drain_rows(a2, gsems[a2])
            scale(a3, a2)
            fire_scatters(a3, a2)

        def body(i, _):
            for k in range(6):
                do_chunk(k, 6 * i + k, i)
            return 0

        lax.fori_loop(0, NCH // 6, body, 0)
        drain_rows(0, gsems[0])
        idx_wait(1)
        drain_rows(1, ssems[1])
        plsc.subcore_barrier()

    def writeback_plain(out_hbm):
        @pl.when(s < NS - 1)
        def _():
            pltpu.sync_copy(
                acc.at[pl.ds(base, ROWS_PER_TILE)],
                out_hbm.at[pl.ds(c * N_NODES + base, ROWS_PER_TILE)],
            )

        @pl.when(s == NS - 1)
        def _():
            pltpu.sync_copy(
                acc.at[pl.ds(base, ROWS_LAST)],
                out_hbm.at[pl.ds(c * N_NODES + base, ROWS_LAST)],
            )
        plsc.subcore_barrier()

    def writeback_sum():
        def sum_chunk(off, n):
            pltpu.sync_copy(e1_hbm.at[pl.ds(c * N_NODES + base + off, n)],
                            r0.at[pl.ds(0, n)])
            pltpu.sync_copy(e2_hbm.at[pl.ds(c * N_NODES + base + off, n)],
                            r1.at[pl.ds(0, n)])

            @plsc.parallel_loop(0, n)
            def _s1(i):
                r0[i, :] = r0[i, :] + r1[i, :]

            pltpu.sync_copy(acc.at[pl.ds(base + off, n)], r1.at[pl.ds(0, n)])

            @plsc.parallel_loop(0, n)
            def _s2(i):
                r0[i, :] = r0[i, :] + r1[i, :]

            pltpu.sync_copy(r0.at[pl.ds(0, n)],
                            esum_hbm.at[pl.ds(c * N_NODES + base + off, n)])

        for z in range(ROWS_PER_TILE // CH):
            sum_chunk(z * CH, CH)

        @pl.when(s < NS - 1)
        def _():
            sum_chunk(9 * CH, ROWS_PER_TILE - 9 * CH)

        @pl.when(s == NS - 1)
        def _():
            sum_chunk(9 * CH, ROWS_LAST - 9 * CH)

    run_layer(allcat_hbm, 2, c)
    writeback_plain(e1_hbm)
    run_layer(e1_hbm, 1, c * N_NODES)
    writeback_plain(e2_hbm)
    run_layer(e2_hbm, 1, c * N_NODES)
    writeback_sum()

    t2 = c * BATCH_USERS + s * 16
    pltpu.sync_copy(uie0_hbm.at[pl.ds(t2, 16)], uidx)
    pltpu.async_copy(allcat_hbm.at[uidx], uta, gsem0).wait()
    pltpu.sync_copy(uies_hbm.at[pl.ds(t2, 16)], uidx)
    pltpu.async_copy(esum_hbm.at[uidx], utb, gsem0).wait()
    for i in range(16):
        uto[i, :] = (uta[i, :] + utb[i, :]) * 0.25
    pltpu.sync_copy(uto, u_hbm.at[pl.ds(t2, 16)])


_BI = 2000
_BU = 64


def _imean_body(ie_ref, ea_ref, eb_ref, o_ref):
    o_ref[...] = (
        ie_ref[...] + jnp.concatenate([ea_ref[...], eb_ref[...]], axis=1)
    ) * 0.25


def _rating_body(u_ref, items_ref, o_ref):
    acc = lax.dot_general(
        u_ref[...], items_ref[...], (((1,), (1,)), ((), ())),
        preferred_element_type=jnp.float32,
    )
    o_ref[...] = 1.0 / (1.0 + jnp.exp(-acc))


def _epilogue(u, esum, item_emb):
    a_off = N_USERS // _BI
    b_off = (N_NODES + N_USERS) // _BI
    items = pl.pallas_call(
        _imean_body,
        grid=(N_ITEMS // _BI,),
        in_specs=[
            pl.BlockSpec((_BI, DIM), lambda i: (i, 0)),
            pl.BlockSpec((_BI, HALF), lambda i: (a_off + i, 0)),
            pl.BlockSpec((_BI, HALF), lambda i: (b_off + i, 0)),
        ],
        out_specs=pl.BlockSpec((_BI, DIM), lambda i: (i, 0)),
        out_shape=jax.ShapeDtypeStruct((N_ITEMS, DIM), jnp.float32),
    )(item_emb, esum, esum)
    return pl.pallas_call(
        _rating_body,
        grid=(BATCH_USERS // _BU,),
        in_specs=[
            pl.BlockSpec((_BU, DIM), lambda i: (i, 0)),
            pl.BlockSpec((N_ITEMS, DIM), lambda i: (0, 0)),
        ],
        out_specs=pl.BlockSpec((_BU, N_ITEMS), lambda i: (i, 0)),
        out_shape=jax.ShapeDtypeStruct((BATCH_USERS, N_ITEMS), jnp.float32),
    )(u, items)


def kernel(users, edge_index, edge_weight, user_emb, item_emb):
    users = users.astype(jnp.int32)
    src = edge_index[0].astype(jnp.int32)
    dst = edge_index[1].astype(jnp.int32)
    w = edge_weight.astype(jnp.float32)

    allcat = jnp.concatenate([user_emb, item_emb], axis=0).reshape(
        2 * N_NODES, HALF
    )

    pad = TOT - N_EDGES
    slack = ((0, 2 * CHB), (0, 0))
    src2d = jnp.pad(jnp.pad(src, (0, pad)).reshape(IDXROWS, 128), slack)
    dst2d = jnp.pad(jnp.pad(dst, (0, pad)).reshape(IDXROWS, 128), slack)
    w2d = jnp.pad(jnp.pad(w, (0, pad)).reshape(IDXROWS, 128), slack)

    uie0 = jnp.concatenate([2 * users, 2 * users + 1])
    uies = jnp.concatenate([users, users + N_NODES])
    e1, e2, esum, u2 = _propagate(src2d, dst2d, w2d, allcat, uie0, uies)
    del e1, e2
    u = jnp.concatenate([u2[:BATCH_USERS], u2[BATCH_USERS:]], axis=1)
    return _epilogue(u, esum, item_emb)
